# baseline (speedup 1.0000x reference)
import functools

import jax
import jax.numpy as jnp
from jax import lax
from jax.experimental import pallas as pl
from jax.experimental.pallas import tpu as pltpu
from jax.experimental.pallas import tpu_sc as plsc

_N = 100000
_K = 64
_NC = 2
_NS = 16
_NW = _NC * _NS
_RW = 3200
_CH = 160
_NCH = 20
_MAXOFF = _N - _CH
_XW = 3328
_P_PAD = 302464


_GD = lax.GatherDimensionNumbers(offset_dims=(), collapsed_slice_dims=(0,),
                                 start_index_map=(0,))


def _bcast_lane(v, j):
    idx = jnp.full((16, 1), j, jnp.int32)
    return lax.gather(v, idx, _GD, (1,),
                      mode=lax.GatherScatterMode.PROMISE_IN_BOUNDS)


def _sc_body(x_hbm, u_hbm, b_hbm, out_hbm,
             xu_v, xt_v, xb_v, ub0, ub1, bb0, bb1, stage,
             s_u0, s_u1, s_b0, s_b1):
    cid = lax.axis_index("c")
    sid = lax.axis_index("s")
    wid = sid * _NC + cid
    r0 = wid * _RW

    def xwin(seg, dst):
        start = seg + r0
        aw = (start // 128) * 128
        pltpu.sync_copy(x_hbm.at[pl.ds(aw, _XW)], dst)
        return start - aw

    dwu = xwin(0, xu_v)
    dwt = xwin(_N, xt_v)
    dwb = xwin(2 * _N, xb_v)

    ubufs = (ub0, ub1)
    bbufs = (bb0, bb1)
    usems = (s_u0, s_u1)
    bsems = (s_b0, s_b1)

    def chunk_off(c):
        return jnp.minimum(r0 + c * _CH, _MAXOFF)

    def start(c, b):
        off = chunk_off(c)
        pltpu.make_async_copy(u_hbm.at[pl.ds(off, _CH), :],
                              ubufs[b], usems[b]).start()
        pltpu.make_async_copy(b_hbm.at[pl.ds(off, _CH), :],
                              bbufs[b], bsems[b]).start()

    def wait(c, b):
        off = chunk_off(c)
        pltpu.make_async_copy(u_hbm.at[pl.ds(off, _CH), :],
                              ubufs[b], usems[b]).wait()
        pltpu.make_async_copy(b_hbm.at[pl.ds(off, _CH), :],
                              bbufs[b], bsems[b]).wait()

    start(0, 0)
    start(1, 1)

    iota = lax.iota(jnp.int32, 16)
    zf = jnp.zeros((16,), jnp.float32)
    accs = tuple(zf for _ in range(16))

    def pair(p, carry):
        for b in range(2):
            c = 2 * p + b
            wait(c, b)
            ubuf = ubufs[b]
            bbuf = bbufs[b]
            loff = chunk_off(c) - r0

            def group(g, gcarry, c=c, ubuf=ubuf, bbuf=bbuf, loff=loff):
                au = list(gcarry[0:4])
                at = list(gcarry[4:8])
                as_ = list(gcarry[8:12])
                aq = list(gcarry[12:16])
                base = g * 16
                lbase = loff + base
                loc = lbase + iota
                valid = (loc >= c * _CH) & ((r0 + loc) < _N)
                xu_g = jnp.where(valid, xu_v[pl.ds(dwu + lbase, 16)], zf)
                xt_g = jnp.where(valid, xt_v[pl.ds(dwt + lbase, 16)], zf)
                xb_g = xb_v[pl.ds(dwb + lbase, 16)]
                bmask = valid & (xb_g == 1.0)
                m_g = jnp.where(bmask, 1.0, zf)

                for j in range(16):
                    row = base + j
                    wu = _bcast_lane(xu_g, j)
                    wt = _bcast_lane(xt_g, j)
                    for k in range(4):
                        au[k] = au[k] + wu * ubuf[row, pl.ds(k * 16, 16)]
                        at[k] = at[k] + wt * bbuf[row, pl.ds(k * 16, 16)]

                def basket(ops):
                    bs_, bq = list(ops[0:4]), list(ops[4:8])
                    for j in range(16):
                        row = base + j
                        m = _bcast_lane(m_g, j)
                        for k in range(4):
                            br = bbuf[row, pl.ds(k * 16, 16)]
                            mb = m * br
                            bs_[k] = bs_[k] + mb
                            bq[k] = bq[k] + mb * br
                    return tuple(bs_) + tuple(bq)

                has_basket = plsc.all_reduce_population_count(bmask)[0] > 0
                res = lax.cond(has_basket, basket, lambda ops: ops,
                               tuple(as_) + tuple(aq))
                as_, aq = list(res[0:4]), list(res[4:8])
                return tuple(au) + tuple(at) + tuple(as_) + tuple(aq)

            carry = lax.fori_loop(0, _CH // 16, group, carry)
            start(c + 2, b)
        return carry

    accs = lax.fori_loop(0, _NCH // 2, pair, accs)
    wait(_NCH, 0)
    wait(_NCH + 1, 1)

    for v in range(16):
        stage[0, pl.ds(v * 16, 16)] = accs[v]
    pltpu.sync_copy(stage, out_hbm.at[wid])


@functools.partial(
    pl.kernel,
    out_type=jax.ShapeDtypeStruct((_NW, 1, 256), jnp.float32),
    mesh=plsc.VectorSubcoreMesh(core_axis_name="c", subcore_axis_name="s"),
    compiler_params=pltpu.CompilerParams(needs_layout_passes=False),
    scratch_types=[
        pltpu.VMEM((_XW,), jnp.float32),
        pltpu.VMEM((_XW,), jnp.float32),
        pltpu.VMEM((_XW,), jnp.float32),
        pltpu.VMEM((_CH, _K), jnp.float32),
        pltpu.VMEM((_CH, _K), jnp.float32),
        pltpu.VMEM((_CH, _K), jnp.float32),
        pltpu.VMEM((_CH, _K), jnp.float32),
        pltpu.VMEM((1, 256), jnp.float32),
        pltpu.SemaphoreType.DMA,
        pltpu.SemaphoreType.DMA,
        pltpu.SemaphoreType.DMA,
        pltpu.SemaphoreType.DMA,
    ],
)
def _sc_partials(x_hbm, u_hbm, b_hbm, out_hbm, *scratch):
    _sc_body(x_hbm, u_hbm, b_hbm, out_hbm, *scratch)


def _fin_body(w0_ref, parts_ref, x2_ref, w2_ref, out_ref):
    parts = parts_ref[...]
    tot = jnp.sum(parts[:, 0, :], axis=0, keepdims=True)
    u = tot[:, 0:64]
    t = tot[:, 64:128]
    s = tot[:, 128:192]
    q = tot[:, 192:256]
    bias = jnp.sum(x2_ref[...] * w2_ref[...])
    u_t = jnp.sum(u * t)
    t_b = jnp.sum(t * s)
    u_b = jnp.sum(u * s)
    bs = 0.5 * (jnp.sum(s * s) - jnp.sum(q))
    y = w0_ref[0, 0] + bias + u_t + t_b + bs + u_b
    out_ref[...] = jnp.reshape(y, (1, 1))


_ROWS2 = _P_PAD // 128


@jax.jit
def _fm(x, w_0, w_bias, u_V, b_V):
    pad = jnp.zeros((_P_PAD - 3 * _N,), jnp.float32)
    xp = jnp.concatenate([x, pad])
    wp = jnp.concatenate([w_bias[:, 0], pad])
    parts = _sc_partials(xp, u_V, b_V)
    x2 = xp.reshape(_ROWS2, 128)
    w2 = wp.reshape(_ROWS2, 128)
    w0 = w_0.reshape(1, 1)
    return pl.pallas_call(
        _fin_body,
        out_shape=jax.ShapeDtypeStruct((1, 1), jnp.float32),
    )(w0, parts, x2, w2)


def kernel(x, delta, pmi, w_0, w_bias, u_V, b_V):
    return _fm(x, w_0, w_bias, u_V, b_V)


# speedup vs baseline: 1.1139x; 1.1139x over previous
import jax
import jax.numpy as jnp
from jax.experimental import pallas as pl
from jax.experimental.pallas import tpu as pltpu

_N = 100000
_K = 64
_BLK = 2000
_NB = _N // _BLK


def _body(w0_ref, xu, xt, xb, wu, wt, wb, uV, bV,
          out_ref, acc_u, acc_t, acc_s, acc_sq, acc_b):
    i = pl.program_id(0)

    @pl.when(i == 0)
    def _init():
        acc_u[...] = jnp.zeros_like(acc_u)
        acc_t[...] = jnp.zeros_like(acc_t)
        acc_s[...] = jnp.zeros_like(acc_s)
        acc_sq[...] = jnp.zeros_like(acc_sq)
        acc_b[...] = jnp.zeros_like(acc_b)

    xu_v = xu[0]
    xt_v = xt[0]
    xb_v = xb[0]
    u_blk = uV[...]
    b_blk = bV[...]
    maskw = (xb_v == 1.0).astype(jnp.float32)

    acc_u[...] += jnp.dot(xu_v, u_blk, preferred_element_type=jnp.float32)
    acc_t[...] += jnp.dot(xt_v, b_blk, preferred_element_type=jnp.float32)
    acc_s[...] += jnp.dot(maskw, b_blk, preferred_element_type=jnp.float32)
    acc_sq[...] += jnp.dot(maskw, b_blk * b_blk,
                           preferred_element_type=jnp.float32)
    wsum = jnp.sum(xu_v * wu[0] + xt_v * wt[0] + xb_v * wb[0])
    acc_b[...] += jnp.reshape(wsum, (1, 1))

    @pl.when(i == _NB - 1)
    def _fin():
        u = acc_u[...]
        t = acc_t[...]
        s = acc_s[...]
        u_t = jnp.sum(u * t)
        t_b = jnp.sum(t * s)
        u_b = jnp.sum(u * s)
        bs = 0.5 * (jnp.sum(s * s) - jnp.sum(acc_sq[...]))
        y = w0_ref[0, 0] + acc_b[0, 0] + u_t + t_b + bs + u_b
        out_ref[...] = jnp.reshape(y, (1, 1))


def _xspec(off):
    return pl.BlockSpec((1, 1, _BLK), lambda i, off=off: (i + off, 0, 0))


_VSPEC = pl.BlockSpec((_BLK, _K), lambda i: (i, 0))


@jax.jit
def _fm(x, w_0, w_bias, u_V, b_V):
    x3 = x.reshape(3 * _NB, 1, _BLK)
    w3 = w_bias.reshape(3 * _NB, 1, _BLK)
    w0 = w_0.reshape(1, 1)
    return pl.pallas_call(
        _body,
        grid=(_NB,),
        in_specs=[
            pl.BlockSpec((1, 1), lambda i: (0, 0)),
            _xspec(0), _xspec(_NB), _xspec(2 * _NB),
            _xspec(0), _xspec(_NB), _xspec(2 * _NB),
            _VSPEC, _VSPEC,
        ],
        out_specs=pl.BlockSpec((1, 1), lambda i: (0, 0)),
        out_shape=jax.ShapeDtypeStruct((1, 1), jnp.float32),
        scratch_shapes=[pltpu.VMEM((1, _K), jnp.float32)] * 4
        + [pltpu.VMEM((1, 1), jnp.float32)],
    )(w0, x3, x3, x3, w3, w3, w3, u_V, b_V)


def kernel(x, delta, pmi, w_0, w_bias, u_V, b_V):
    return _fm(x, w_0, w_bias, u_V, b_V)
